# Initial kernel scaffold; baseline (speedup 1.0000x reference)
import jax
import jax.numpy as jnp
from jax.experimental import pallas as pl


def kernel(species, embedding):
    raise NotImplementedError("write your pallas kernel here")


# speedup vs baseline: 2.7232x; 2.7232x over previous
import functools

import jax
import jax.numpy as jnp
from jax import lax
from jax.experimental import pallas as pl
from jax.experimental.pallas import tpu as pltpu
from jax.experimental.pallas import tpu_sc as plsc

NUM_FEATURES = 16

_info = plsc.get_sparse_core_info()
_NC, _NS = _info.num_cores, _info.num_subcores
_NW = _NC * _NS


@functools.lru_cache(maxsize=None)
def _make_gather(n: int, chunk: int):
    assert n % (_NW * chunk) == 0 and chunk % 8 == 0
    per_w = n // _NW
    n_chunks = per_w // chunk
    mesh = plsc.VectorSubcoreMesh(core_axis_name="c", subcore_axis_name="s")

    @functools.partial(
        pl.kernel,
        out_type=jax.ShapeDtypeStruct((n, NUM_FEATURES), jnp.float32),
        mesh=mesh,
        scratch_types=[
            pltpu.VMEM((chunk,), jnp.int32),
            pltpu.VMEM((chunk, NUM_FEATURES), jnp.float32),
            pltpu.SemaphoreType.DMA,
        ],
        compiler_params=pltpu.CompilerParams(use_tc_tiling_on_sc=False),
    )
    def gather(idx_hbm, table_hbm, out_hbm, idx_v, rows_v, sem):
        wid = lax.axis_index("s") * _NC + lax.axis_index("c")
        base = wid * per_w

        def body(g, carry):
            off = base + g * chunk
            pltpu.sync_copy(idx_hbm.at[pl.ds(off, chunk)], idx_v)
            pltpu.async_copy(table_hbm.at[idx_v], rows_v, sem).wait()
            pltpu.sync_copy(rows_v, out_hbm.at[pl.ds(off, chunk)])
            return carry

        lax.fori_loop(0, n_chunks, body, 0)

    return gather


def kernel(species, embedding):
    b, s = species.shape
    n = b * s
    flat = species.reshape(n).astype(jnp.int32)
    out = _make_gather(n, 2048)(flat, embedding)
    return out.reshape(b, s, NUM_FEATURES)


# speedup vs baseline: 2.7583x; 1.0129x over previous
import functools

import jax
import jax.numpy as jnp
from jax import lax
from jax.experimental import pallas as pl
from jax.experimental.pallas import tpu as pltpu
from jax.experimental.pallas import tpu_sc as plsc

NUM_FEATURES = 16

_info = plsc.get_sparse_core_info()
_NC, _NS = _info.num_cores, _info.num_subcores
_NW = _NC * _NS


@functools.lru_cache(maxsize=None)
def _make_gather(n: int, chunk: int, nbuf: int):
    assert n % (_NW * chunk) == 0 and chunk % 8 == 0
    per_w = n // _NW
    n_chunks = per_w // chunk
    assert n_chunks % nbuf == 0 and n_chunks >= nbuf
    mesh = plsc.VectorSubcoreMesh(core_axis_name="c", subcore_axis_name="s")

    @functools.partial(
        pl.kernel,
        out_type=jax.ShapeDtypeStruct((n, NUM_FEATURES), jnp.float32),
        mesh=mesh,
        scratch_types=[
            pltpu.VMEM((per_w,), jnp.int32),
            pltpu.VMEM((nbuf, chunk, NUM_FEATURES), jnp.float32),
            [pltpu.SemaphoreType.DMA] * nbuf,
            [pltpu.SemaphoreType.DMA] * nbuf,
        ],
        compiler_params=pltpu.CompilerParams(use_tc_tiling_on_sc=False),
    )
    def gather(idx_hbm, table_hbm, out_hbm, idx_v, rows_v, gsem, ssem):
        wid = lax.axis_index("s") * _NC + lax.axis_index("c")
        base = wid * per_w

        pltpu.sync_copy(idx_hbm.at[pl.ds(base, per_w)], idx_v)
        for b in range(nbuf):
            pltpu.async_copy(
                table_hbm.at[idx_v.at[pl.ds(b * chunk, chunk)]],
                rows_v.at[b],
                gsem[b],
            )

        @pl.loop(0, n_chunks, step=nbuf)
        def _(gg):
            for b in range(nbuf):
                g = gg + b
                pltpu.make_async_copy(
                    table_hbm.at[idx_v.at[pl.ds(g * chunk, chunk)]],
                    rows_v.at[b],
                    gsem[b],
                ).wait()
                pltpu.async_copy(
                    rows_v.at[b],
                    out_hbm.at[pl.ds(base + g * chunk, chunk)],
                    ssem[b],
                )

                @pl.when(g + nbuf < n_chunks)
                def _():
                    pltpu.make_async_copy(
                        rows_v.at[b], out_hbm.at[pl.ds(base, chunk)], ssem[b]
                    ).wait()
                    pltpu.async_copy(
                        table_hbm.at[idx_v.at[pl.ds((g + nbuf) * chunk, chunk)]],
                        rows_v.at[b],
                        gsem[b],
                    )

        for b in range(nbuf):
            pltpu.make_async_copy(
                rows_v.at[b], out_hbm.at[pl.ds(base, chunk)], ssem[b]
            ).wait()

    return gather


def kernel(species, embedding):
    b, s = species.shape
    n = b * s
    flat = species.reshape(n).astype(jnp.int32)
    out = _make_gather(n, 1600, 2)(flat, embedding)
    return out.reshape(b, s, NUM_FEATURES)


# speedup vs baseline: 25.0897x; 9.0961x over previous
import functools

import jax
import jax.numpy as jnp
from jax import lax
from jax.experimental import pallas as pl
from jax.experimental.pallas import tpu as pltpu
from jax.experimental.pallas import tpu_sc as plsc

NUM_FEATURES = 16

_info = plsc.get_sparse_core_info()
_NC, _NS = _info.num_cores, _info.num_subcores


@functools.lru_cache(maxsize=None)
def _make_gather(s_dim: int, b_dim: int, vocab: int, chunk: int):
    half = b_dim // _NC
    k_per_s = half // chunk
    n_chunks = s_dim * k_per_s
    assert half % chunk == 0 and chunk % 16 == 0 and n_chunks % 2 == 0
    mesh = plsc.VectorSubcoreMesh(core_axis_name="c", subcore_axis_name="s")

    @functools.partial(
        pl.kernel,
        out_type=jax.ShapeDtypeStruct((s_dim, NUM_FEATURES, b_dim), jnp.float32),
        mesh=mesh,
        scratch_types=[
            pltpu.VMEM((vocab,), jnp.float32),
            [pltpu.VMEM((chunk,), jnp.int32)] * 2,
            [pltpu.VMEM((chunk,), jnp.float32)] * 2,
            [pltpu.SemaphoreType.DMA] * 2,
            [pltpu.SemaphoreType.DMA] * 2,
        ],
        compiler_params=pltpu.CompilerParams(
            use_tc_tiling_on_sc=True, needs_layout_passes=False
        ),
    )
    def gather(sp_hbm, emb_hbm, out_hbm, row_v, idx_v, out_v, isem, osem):
        f = lax.axis_index("s")
        b0 = lax.axis_index("c") * half

        pltpu.sync_copy(emb_hbm.at[f], row_v)

        def idx_src(t):
            s = t // k_per_s
            k = t % k_per_s
            return sp_hbm.at[s, pl.ds(b0 + k * chunk, chunk)]

        def out_dst(t):
            s = t // k_per_s
            k = t % k_per_s
            return out_hbm.at[s, f, pl.ds(b0 + k * chunk, chunk)]

        for b in range(2):
            pltpu.async_copy(idx_src(b), idx_v[b], isem[b])

        @pl.loop(0, n_chunks, step=2)
        def _(tt):
            for b in range(2):
                t = tt + b
                pltpu.make_async_copy(idx_src(t), idx_v[b], isem[b]).wait()

                @pl.when(t >= 2)
                def _():
                    pltpu.make_async_copy(out_v[b], out_dst(t), osem[b]).wait()

                @pl.loop(0, chunk // 16)
                def _(j):
                    idx = idx_v[b][pl.ds(j * 16, 16)]
                    out_v[b][pl.ds(j * 16, 16)] = plsc.load_gather(
                        row_v, [idx]
                    )

                pltpu.async_copy(out_v[b], out_dst(t), osem[b])

                @pl.when(t + 2 < n_chunks)
                def _():
                    pltpu.async_copy(idx_src(t + 2), idx_v[b], isem[b])

        for b in range(2):
            pltpu.make_async_copy(
                out_v[b], out_dst(n_chunks - 2 + b), osem[b]
            ).wait()

    return gather


def kernel(species, embedding):
    b_dim, s_dim = species.shape
    vocab, feat = embedding.shape
    sp_t = species.T.astype(jnp.int32)
    emb_t = embedding.T
    out_t = _make_gather(s_dim, b_dim, vocab, 4096)(sp_t, emb_t)
    return jnp.transpose(out_t, (2, 0, 1))


# speedup vs baseline: 49.8396x; 1.9865x over previous
import functools

import jax
import jax.numpy as jnp
from jax import lax
from jax.experimental import pallas as pl
from jax.experimental.pallas import tpu as pltpu
from jax.experimental.pallas import tpu_sc as plsc

NUM_FEATURES = 16

_info = plsc.get_sparse_core_info()
_NC, _NS = _info.num_cores, _info.num_subcores


@functools.lru_cache(maxsize=None)
def _make_gather(s_dim: int, b_dim: int, vocab: int, chunk: int):
    half = b_dim // _NC
    k_per_s = half // chunk
    n_chunks = s_dim * k_per_s
    assert half % chunk == 0 and chunk % 16 == 0 and n_chunks % 2 == 0
    mesh = plsc.VectorSubcoreMesh(core_axis_name="c", subcore_axis_name="s")

    @functools.partial(
        pl.kernel,
        out_type=jax.ShapeDtypeStruct((s_dim, NUM_FEATURES, b_dim), jnp.float32),
        mesh=mesh,
        scratch_types=[
            pltpu.VMEM((vocab,), jnp.float32),
            [pltpu.VMEM((chunk,), jnp.int32)] * 2,
            [pltpu.VMEM((chunk,), jnp.float32)] * 2,
            [pltpu.SemaphoreType.DMA] * 2,
            [pltpu.SemaphoreType.DMA] * 2,
        ],
        compiler_params=pltpu.CompilerParams(
            use_tc_tiling_on_sc=True, needs_layout_passes=False
        ),
    )
    def gather(sp_hbm, emb_hbm, out_hbm, row_v, idx_v, out_v, isem, osem):
        f = lax.axis_index("s")
        b0 = lax.axis_index("c") * half

        pltpu.sync_copy(emb_hbm.at[f], row_v)

        def idx_src(t):
            s = t // k_per_s
            k = t % k_per_s
            return sp_hbm.at[s, pl.ds(b0 + k * chunk, chunk)]

        def out_dst(t):
            s = t // k_per_s
            k = t % k_per_s
            return out_hbm.at[s, f, pl.ds(b0 + k * chunk, chunk)]

        for b in range(2):
            pltpu.async_copy(idx_src(b), idx_v[b], isem[b])

        @pl.loop(0, n_chunks, step=2)
        def _(tt):
            for b in range(2):
                t = tt + b
                pltpu.make_async_copy(idx_src(t), idx_v[b], isem[b]).wait()

                @pl.when(t >= 2)
                def _():
                    pltpu.make_async_copy(out_v[b], out_dst(t), osem[b]).wait()

                @plsc.parallel_loop(0, chunk, step=16, unroll=8)
                def _(j):
                    idx = idx_v[b][pl.ds(j, 16)]
                    out_v[b][pl.ds(j, 16)] = plsc.load_gather(
                        row_v, [idx]
                    )

                pltpu.async_copy(out_v[b], out_dst(t), osem[b])

                @pl.when(t + 2 < n_chunks)
                def _():
                    pltpu.async_copy(idx_src(t + 2), idx_v[b], isem[b])

        for b in range(2):
            pltpu.make_async_copy(
                out_v[b], out_dst(n_chunks - 2 + b), osem[b]
            ).wait()

    return gather


def kernel(species, embedding):
    b_dim, s_dim = species.shape
    vocab, feat = embedding.shape
    sp_t = species.T.astype(jnp.int32)
    emb_t = embedding.T
    out_t = _make_gather(s_dim, b_dim, vocab, 4096)(sp_t, emb_t)
    return jnp.transpose(out_t, (2, 0, 1))


# speedup vs baseline: 49.8797x; 1.0008x over previous
import functools

import jax
import jax.numpy as jnp
from jax import lax
from jax.experimental import pallas as pl
from jax.experimental.pallas import tpu as pltpu
from jax.experimental.pallas import tpu_sc as plsc

NUM_FEATURES = 16

_info = plsc.get_sparse_core_info()
_NC, _NS = _info.num_cores, _info.num_subcores


@functools.lru_cache(maxsize=None)
def _make_gather(s_dim: int, b_dim: int, vocab: int, chunk: int):
    half = b_dim // _NC
    k_per_s = half // chunk
    n_chunks = s_dim * k_per_s
    assert half % chunk == 0 and chunk % 16 == 0 and n_chunks % 2 == 0
    mesh = plsc.VectorSubcoreMesh(core_axis_name="c", subcore_axis_name="s")

    @functools.partial(
        pl.kernel,
        out_type=jax.ShapeDtypeStruct((s_dim, NUM_FEATURES, b_dim), jnp.float32),
        mesh=mesh,
        scratch_types=[
            pltpu.VMEM((vocab,), jnp.float32),
            [pltpu.VMEM((chunk,), jnp.int32)] * 2,
            [pltpu.VMEM((chunk,), jnp.float32)] * 2,
            [pltpu.SemaphoreType.DMA] * 2,
            [pltpu.SemaphoreType.DMA] * 2,
        ],
        compiler_params=pltpu.CompilerParams(
            use_tc_tiling_on_sc=True, needs_layout_passes=False
        ),
    )
    def gather(sp_hbm, emb_hbm, out_hbm, row_v, idx_v, out_v, isem, osem):
        f = lax.axis_index("s")
        b0 = lax.axis_index("c") * half

        pltpu.sync_copy(emb_hbm.at[f], row_v)

        def idx_src(t):
            s = t // k_per_s
            k = t % k_per_s
            return sp_hbm.at[s, pl.ds(b0 + k * chunk, chunk)]

        def out_dst(t):
            s = t // k_per_s
            k = t % k_per_s
            return out_hbm.at[s, f, pl.ds(b0 + k * chunk, chunk)]

        for b in range(2):
            pltpu.async_copy(idx_src(b), idx_v[b], isem[b])

        @pl.loop(0, n_chunks, step=2)
        def _(tt):
            for b in range(2):
                t = tt + b
                pltpu.make_async_copy(idx_src(t), idx_v[b], isem[b]).wait()

                @pl.when(t >= 2)
                def _():
                    pltpu.make_async_copy(out_v[b], out_dst(t), osem[b]).wait()

                @plsc.parallel_loop(0, chunk, step=16, unroll=16)
                def _(j):
                    idx = idx_v[b][pl.ds(j, 16)]
                    out_v[b][pl.ds(j, 16)] = plsc.load_gather(
                        row_v, [idx]
                    )

                pltpu.async_copy(out_v[b], out_dst(t), osem[b])

                @pl.when(t + 2 < n_chunks)
                def _():
                    pltpu.async_copy(idx_src(t + 2), idx_v[b], isem[b])

        for b in range(2):
            pltpu.make_async_copy(
                out_v[b], out_dst(n_chunks - 2 + b), osem[b]
            ).wait()

    return gather


def kernel(species, embedding):
    b_dim, s_dim = species.shape
    vocab, feat = embedding.shape
    sp_t = species.T.astype(jnp.int32)
    emb_t = embedding.T
    out_t = _make_gather(s_dim, b_dim, vocab, 4096)(sp_t, emb_t)
    return jnp.transpose(out_t, (2, 0, 1))
